# Initial kernel scaffold; baseline (speedup 1.0000x reference)
import functools

import jax
import jax.numpy as jnp
from jax import lax
from jax.experimental import pallas as pl
from jax.experimental.pallas import tpu as pltpu
from jax.experimental.pallas import tpu_sc as plsc

B = 16384
MW = 20
MA = 10
UD = 64
AD = 32
DIM = 256
XCOLS = 2 * UD + 3 * AD

G = 8
L = 16


def _sc_embed(ti, tg, ii, ci, oi, wt, tt, it, ct):
    mesh = plsc.VectorSubcoreMesh(core_axis_name="c", subcore_axis_name="s")
    nw = mesh.num_cores * mesh.num_subcores
    R = B // nw
    NG = R // G

    @functools.partial(
        pl.kernel,
        out_type=jax.ShapeDtypeStruct((B, XCOLS), jnp.float32),
        mesh=mesh,
        scratch_types=[
            pltpu.VMEM((R * MW,), jnp.int32),
            pltpu.VMEM((R,), jnp.int32),
            pltpu.VMEM((R * MA,), jnp.int32),
            pltpu.VMEM((R * MA,), jnp.int32),
            pltpu.VMEM((R * MW,), jnp.int32),
            pltpu.VMEM((2, G * MW, UD), jnp.float32),
            pltpu.VMEM((2, G, AD), jnp.float32),
            pltpu.VMEM((2, G * MA, AD), jnp.float32),
            pltpu.VMEM((2, G * MA, AD), jnp.float32),
            pltpu.VMEM((2, G * MW, UD), jnp.float32),
            pltpu.VMEM((2, G, XCOLS), jnp.float32),
            pltpu.SemaphoreType.DMA,
            pltpu.SemaphoreType.DMA,
        ],
    )
    def k(ti_h, tg_h, ii_h, ci_h, oi_h, wt_h, tt_h, it_h, ct_h, x_h,
          xt, xg, xi, xc, xo, rt, rg, ri, rc, ro, xout, sem_g, sem_s):
        wid = lax.axis_index("s") * mesh.num_cores + lax.axis_index("c")
        base = wid * R

        pltpu.sync_copy(ti_h.at[pl.ds(base * MW, R * MW)], xt)
        pltpu.sync_copy(tg_h.at[pl.ds(base, R)], xg)
        pltpu.sync_copy(ii_h.at[pl.ds(base * MA, R * MA)], xi)
        pltpu.sync_copy(ci_h.at[pl.ds(base * MA, R * MA)], xc)
        pltpu.sync_copy(oi_h.at[pl.ds(base * MW, R * MW)], xo)

        H = G * MW // 2

        def gather_pairs(g):
            slot = g % 2
            tw = g * G * MW
            ta = g * G * MA
            return [
                (wt_h.at[xt.at[pl.ds(tw, H)]], rt.at[slot, pl.ds(0, H)]),
                (wt_h.at[xt.at[pl.ds(tw + H, H)]], rt.at[slot, pl.ds(H, H)]),
                (wt_h.at[xo.at[pl.ds(tw, H)]], ro.at[slot, pl.ds(0, H)]),
                (wt_h.at[xo.at[pl.ds(tw + H, H)]], ro.at[slot, pl.ds(H, H)]),
                (it_h.at[xi.at[pl.ds(ta, G * MA)]], ri.at[slot]),
                (ct_h.at[xc.at[pl.ds(ta, G * MA)]], rc.at[slot]),
                (tt_h.at[xg.at[pl.ds(g * G, G)]], rg.at[slot]),
            ]

        def issue_g(g):
            for s, d in gather_pairs(g):
                pltpu.async_copy(s, d, sem_g)

        def wait_g(g):
            for s, d in gather_pairs(g):
                pltpu.make_async_copy(s, d, sem_g).wait()

        def store_pair(g):
            slot = g % 2
            return xout.at[slot], x_h.at[pl.ds(base + g * G, G)]

        def issue_s(g):
            s, d = store_pair(g)
            pltpu.async_copy(s, d, sem_s)

        def wait_s(g):
            s, d = store_pair(g)
            pltpu.make_async_copy(s, d, sem_s).wait()

        def compute(g):
            slot = g % 2

            def body_j(j, carry):
                for c in range(UD // L):
                    sl = pl.ds(c * L, L)
                    acc = rt[slot, j * MW, sl]
                    for r in range(1, MW):
                        acc = acc + rt[slot, j * MW + r, sl]
                    xout[slot, j, pl.ds(c * L, L)] = acc * (1.0 / MW)
                for c in range(AD // L):
                    xout[slot, j, pl.ds(UD + c * L, L)] = rg[slot, j, pl.ds(c * L, L)]
                for c in range(AD // L):
                    sl = pl.ds(c * L, L)
                    acc = ri[slot, j * MA, sl]
                    for r in range(1, MA):
                        acc = acc + ri[slot, j * MA + r, sl]
                    xout[slot, j, pl.ds(UD + AD + c * L, L)] = acc * (1.0 / MA)
                for c in range(AD // L):
                    sl = pl.ds(c * L, L)
                    acc = rc[slot, j * MA, sl]
                    for r in range(1, MA):
                        acc = acc + rc[slot, j * MA + r, sl]
                    xout[slot, j, pl.ds(UD + 2 * AD + c * L, L)] = acc * (1.0 / MA)
                for c in range(UD // L):
                    sl = pl.ds(c * L, L)
                    acc = ro[slot, j * MW, sl]
                    for r in range(1, MW):
                        acc = acc + ro[slot, j * MW + r, sl]
                    xout[slot, j, pl.ds(UD + 3 * AD + c * L, L)] = acc * (1.0 / MW)
                return carry

            lax.fori_loop(0, G, body_j, 0)

        issue_g(0)

        def loop_body(g, carry):
            issue_g(g)
            wait_g(g - 1)

            @pl.when(g >= 3)
            def _():
                wait_s(g - 3)

            compute(g - 1)
            issue_s(g - 1)
            return carry

        lax.fori_loop(1, NG, loop_body, 0)

        wait_g(NG - 1)
        wait_s(NG - 3)
        compute(NG - 1)
        issue_s(NG - 1)
        wait_s(NG - 2)
        wait_s(NG - 1)

    return k(ti, tg, ii, ci, oi, wt, tt, it, ct)


def _tc_head(x, coords, w_main, w_coords, b):
    BM = 2048

    def body(x_ref, c_ref, w_ref, wc_ref, b_ref, o_ref):
        acc = jnp.dot(x_ref[...], w_ref[...], preferred_element_type=jnp.float32)
        c = c_ref[...]
        acc = acc + c[:, 0:1] * wc_ref[0:1, :]
        acc = acc + c[:, 1:2] * wc_ref[1:2, :]
        acc = acc + c[:, 2:3] * wc_ref[2:3, :]
        o_ref[...] = acc + b_ref[...]

    return pl.pallas_call(
        body,
        grid=(B // BM,),
        in_specs=[
            pl.BlockSpec((BM, XCOLS), lambda i: (i, 0)),
            pl.BlockSpec((BM, 3), lambda i: (i, 0)),
            pl.BlockSpec((XCOLS, DIM), lambda i: (0, 0)),
            pl.BlockSpec((3, DIM), lambda i: (0, 0)),
            pl.BlockSpec((1, DIM), lambda i: (0, 0)),
        ],
        out_specs=pl.BlockSpec((BM, DIM), lambda i: (i, 0)),
        out_shape=jax.ShapeDtypeStruct((B, DIM), jnp.float32),
    )(x, coords, w_main, w_coords, b)


def kernel(text_tokens, tags, id_tokens, class_tokens, other_tokens, coords,
           word_table, tag_table, id_table, class_table, fc_W, fc_b):
    ti = text_tokens.astype(jnp.int32).reshape(-1)
    tg = tags.astype(jnp.int32)
    ii = id_tokens.astype(jnp.int32).reshape(-1)
    ci = class_tokens.astype(jnp.int32).reshape(-1)
    oi = other_tokens.astype(jnp.int32).reshape(-1)
    x = _sc_embed(ti, tg, ii, ci, oi, word_table, tag_table, id_table,
                  class_table)
    return _tc_head(x, coords, fc_W[:XCOLS], fc_W[XCOLS:], fc_b.reshape(1, -1))


# speedup vs baseline: 12.1375x; 12.1375x over previous
import functools

import jax
import jax.numpy as jnp
from jax import lax
from jax.experimental import pallas as pl
from jax.experimental.pallas import tpu as pltpu
from jax.experimental.pallas import tpu_sc as plsc

B = 16384
MW = 20
MA = 10
UD = 64
AD = 32
DIM = 256
XCOLS = 2 * UD + 3 * AD

G = 8
L = 16


def _sc_embed(ti, tg, ii, ci, oi, wt, tt, it, ct):
    mesh = plsc.VectorSubcoreMesh(core_axis_name="c", subcore_axis_name="s")
    nw = mesh.num_cores * mesh.num_subcores
    R = B // nw
    NG = R // G

    @functools.partial(
        pl.kernel,
        out_type=jax.ShapeDtypeStruct((B, XCOLS), jnp.float32),
        mesh=mesh,
        scratch_types=[
            pltpu.VMEM((R * MW,), jnp.int32),
            pltpu.VMEM((R,), jnp.int32),
            pltpu.VMEM((R * MA,), jnp.int32),
            pltpu.VMEM((R * MA,), jnp.int32),
            pltpu.VMEM((R * MW,), jnp.int32),
            pltpu.VMEM((2, G * MW, UD), jnp.float32),
            pltpu.VMEM((2, G, AD), jnp.float32),
            pltpu.VMEM((2, G * MA, AD), jnp.float32),
            pltpu.VMEM((2, G * MA, AD), jnp.float32),
            pltpu.VMEM((2, G * MW, UD), jnp.float32),
            pltpu.VMEM((2, G, XCOLS), jnp.float32),
            pltpu.SemaphoreType.DMA,
            pltpu.SemaphoreType.DMA,
        ],
        compiler_params=pltpu.CompilerParams(use_tc_tiling_on_sc=False),
    )
    def k(ti_h, tg_h, ii_h, ci_h, oi_h, wt_h, tt_h, it_h, ct_h, x_h,
          xt, xg, xi, xc, xo, rt, rg, ri, rc, ro, xout, sem_g, sem_s):
        wid = lax.axis_index("s") * mesh.num_cores + lax.axis_index("c")
        base = wid * R

        pltpu.sync_copy(ti_h.at[pl.ds(base * MW, R * MW)], xt)
        pltpu.sync_copy(tg_h.at[pl.ds(base, R)], xg)
        pltpu.sync_copy(ii_h.at[pl.ds(base * MA, R * MA)], xi)
        pltpu.sync_copy(ci_h.at[pl.ds(base * MA, R * MA)], xc)
        pltpu.sync_copy(oi_h.at[pl.ds(base * MW, R * MW)], xo)

        H = G * MW // 2

        def gather_pairs(g):
            slot = g % 2
            tw = g * G * MW
            ta = g * G * MA
            return [
                (wt_h.at[xt.at[pl.ds(tw, H)]], rt.at[slot, pl.ds(0, H)]),
                (wt_h.at[xt.at[pl.ds(tw + H, H)]], rt.at[slot, pl.ds(H, H)]),
                (wt_h.at[xo.at[pl.ds(tw, H)]], ro.at[slot, pl.ds(0, H)]),
                (wt_h.at[xo.at[pl.ds(tw + H, H)]], ro.at[slot, pl.ds(H, H)]),
                (it_h.at[xi.at[pl.ds(ta, G * MA)]], ri.at[slot]),
                (ct_h.at[xc.at[pl.ds(ta, G * MA)]], rc.at[slot]),
                (tt_h.at[xg.at[pl.ds(g * G, G)]], rg.at[slot]),
            ]

        def issue_g(g):
            for s, d in gather_pairs(g):
                pltpu.async_copy(s, d, sem_g)

        def wait_g(g):
            for s, d in gather_pairs(g):
                pltpu.make_async_copy(s, d, sem_g).wait()

        def store_pair(g):
            slot = g % 2
            return xout.at[slot], x_h.at[pl.ds(base + g * G, G)]

        def issue_s(g):
            s, d = store_pair(g)
            pltpu.async_copy(s, d, sem_s)

        def wait_s(g):
            s, d = store_pair(g)
            pltpu.make_async_copy(s, d, sem_s).wait()

        def compute(g):
            slot = g % 2

            def body_j(j, carry):
                for c in range(UD // L):
                    sl = pl.ds(c * L, L)
                    acc = rt[slot, j * MW, sl]
                    for r in range(1, MW):
                        acc = acc + rt[slot, j * MW + r, sl]
                    xout[slot, j, pl.ds(c * L, L)] = acc * (1.0 / MW)
                for c in range(AD // L):
                    xout[slot, j, pl.ds(UD + c * L, L)] = rg[slot, j, pl.ds(c * L, L)]
                for c in range(AD // L):
                    sl = pl.ds(c * L, L)
                    acc = ri[slot, j * MA, sl]
                    for r in range(1, MA):
                        acc = acc + ri[slot, j * MA + r, sl]
                    xout[slot, j, pl.ds(UD + AD + c * L, L)] = acc * (1.0 / MA)
                for c in range(AD // L):
                    sl = pl.ds(c * L, L)
                    acc = rc[slot, j * MA, sl]
                    for r in range(1, MA):
                        acc = acc + rc[slot, j * MA + r, sl]
                    xout[slot, j, pl.ds(UD + 2 * AD + c * L, L)] = acc * (1.0 / MA)
                for c in range(UD // L):
                    sl = pl.ds(c * L, L)
                    acc = ro[slot, j * MW, sl]
                    for r in range(1, MW):
                        acc = acc + ro[slot, j * MW + r, sl]
                    xout[slot, j, pl.ds(UD + 3 * AD + c * L, L)] = acc * (1.0 / MW)
                return carry

            lax.fori_loop(0, G, body_j, 0)

        issue_g(0)

        def loop_body(g, carry):
            issue_g(g)
            wait_g(g - 1)

            @pl.when(g >= 3)
            def _():
                wait_s(g - 3)

            compute(g - 1)
            issue_s(g - 1)
            return carry

        lax.fori_loop(1, NG, loop_body, 0)

        wait_g(NG - 1)
        wait_s(NG - 3)
        compute(NG - 1)
        issue_s(NG - 1)
        wait_s(NG - 2)
        wait_s(NG - 1)

    return k(ti, tg, ii, ci, oi, wt, tt, it, ct)


def _tc_head(x, coords, w_main, w_coords, b):
    BM = 2048

    def body(x_ref, c_ref, w_ref, wc_ref, b_ref, o_ref):
        acc = jnp.dot(x_ref[...], w_ref[...], preferred_element_type=jnp.float32)
        c = c_ref[...]
        acc = acc + c[:, 0:1] * wc_ref[0:1, :]
        acc = acc + c[:, 1:2] * wc_ref[1:2, :]
        acc = acc + c[:, 2:3] * wc_ref[2:3, :]
        o_ref[...] = acc + b_ref[...]

    return pl.pallas_call(
        body,
        grid=(B // BM,),
        in_specs=[
            pl.BlockSpec((BM, XCOLS), lambda i: (i, 0)),
            pl.BlockSpec((BM, 3), lambda i: (i, 0)),
            pl.BlockSpec((XCOLS, DIM), lambda i: (0, 0)),
            pl.BlockSpec((3, DIM), lambda i: (0, 0)),
            pl.BlockSpec((1, DIM), lambda i: (0, 0)),
        ],
        out_specs=pl.BlockSpec((BM, DIM), lambda i: (i, 0)),
        out_shape=jax.ShapeDtypeStruct((B, DIM), jnp.float32),
    )(x, coords, w_main, w_coords, b)


def kernel(text_tokens, tags, id_tokens, class_tokens, other_tokens, coords,
           word_table, tag_table, id_table, class_table, fc_W, fc_b):
    ti = text_tokens.astype(jnp.int32).reshape(-1)
    tg = tags.astype(jnp.int32)
    ii = id_tokens.astype(jnp.int32).reshape(-1)
    ci = class_tokens.astype(jnp.int32).reshape(-1)
    oi = other_tokens.astype(jnp.int32).reshape(-1)
    x = _sc_embed(ti, tg, ii, ci, oi, word_table, tag_table, id_table,
                  class_table)
    return _tc_head(x, coords, fc_W[:XCOLS], fc_W[XCOLS:], fc_b.reshape(1, -1))


# speedup vs baseline: 12.6519x; 1.0424x over previous
import functools

import jax
import jax.numpy as jnp
from jax import lax
from jax.experimental import pallas as pl
from jax.experimental.pallas import tpu as pltpu
from jax.experimental.pallas import tpu_sc as plsc

B = 16384
MW = 20
MA = 10
UD = 64
AD = 32
DIM = 256
XCOLS = 2 * UD + 3 * AD

G = 8
L = 16

OFF_TEXT = 0
OFF_OTHER = B * MW
OFF_ID = 2 * B * MW
OFF_CLASS = 2 * B * MW + B * MA
OFF_TAG = 2 * B * MW + 2 * B * MA


def _sc_embed(tok, wt, tt, it, ct):
    mesh = plsc.VectorSubcoreMesh(core_axis_name="c", subcore_axis_name="s")
    nw = mesh.num_cores * mesh.num_subcores
    R = B // nw
    NG = R // G

    @functools.partial(
        pl.kernel,
        out_type=(
            jax.ShapeDtypeStruct((B, 128), jnp.float32),
            jax.ShapeDtypeStruct((B, 128), jnp.float32),
        ),
        mesh=mesh,
        scratch_types=[
            pltpu.VMEM((R * MW,), jnp.int32),
            pltpu.VMEM((R,), jnp.int32),
            pltpu.VMEM((R * MA,), jnp.int32),
            pltpu.VMEM((R * MA,), jnp.int32),
            pltpu.VMEM((R * MW,), jnp.int32),
            pltpu.VMEM((2, G * MW, UD), jnp.float32),
            pltpu.VMEM((2, G, AD), jnp.float32),
            pltpu.VMEM((2, G * MA, AD), jnp.float32),
            pltpu.VMEM((2, G * MA, AD), jnp.float32),
            pltpu.VMEM((2, G * MW, UD), jnp.float32),
            pltpu.VMEM((2, G, 128), jnp.float32),
            pltpu.VMEM((2, G, 128), jnp.float32),
            pltpu.SemaphoreType.DMA,
            pltpu.SemaphoreType.DMA,
        ],
        compiler_params=pltpu.CompilerParams(use_tc_tiling_on_sc=False),
    )
    def k(tok_h, wt_h, tt_h, it_h, ct_h, x1_h, x2_h,
          xt, xg, xi, xc, xo, rt, rg, ri, rc, ro, xo1, xo2, sem_g, sem_s):
        wid = lax.axis_index("s") * mesh.num_cores + lax.axis_index("c")
        base = wid * R

        pltpu.sync_copy(tok_h.at[pl.ds(OFF_TEXT + base * MW, R * MW)], xt)
        pltpu.sync_copy(tok_h.at[pl.ds(OFF_TAG + base, R)], xg)
        pltpu.sync_copy(tok_h.at[pl.ds(OFF_ID + base * MA, R * MA)], xi)
        pltpu.sync_copy(tok_h.at[pl.ds(OFF_CLASS + base * MA, R * MA)], xc)
        pltpu.sync_copy(tok_h.at[pl.ds(OFF_OTHER + base * MW, R * MW)], xo)

        zero = jnp.zeros((L,), jnp.float32)
        for s in range(2):
            for j in range(G):
                xo2[s, j, pl.ds(96, L)] = zero
                xo2[s, j, pl.ds(96 + L, L)] = zero

        H = G * MW // 2

        def gather_pairs(g):
            slot = g % 2
            tw = g * G * MW
            ta = g * G * MA
            return [
                (wt_h.at[xt.at[pl.ds(tw, H)]], rt.at[slot, pl.ds(0, H)]),
                (wt_h.at[xt.at[pl.ds(tw + H, H)]], rt.at[slot, pl.ds(H, H)]),
                (wt_h.at[xo.at[pl.ds(tw, H)]], ro.at[slot, pl.ds(0, H)]),
                (wt_h.at[xo.at[pl.ds(tw + H, H)]], ro.at[slot, pl.ds(H, H)]),
                (it_h.at[xi.at[pl.ds(ta, G * MA)]], ri.at[slot]),
                (ct_h.at[xc.at[pl.ds(ta, G * MA)]], rc.at[slot]),
                (tt_h.at[xg.at[pl.ds(g * G, G)]], rg.at[slot]),
            ]

        def issue_g(g):
            for s, d in gather_pairs(g):
                pltpu.async_copy(s, d, sem_g)

        def wait_g(g):
            for s, d in gather_pairs(g):
                pltpu.make_async_copy(s, d, sem_g).wait()

        def store_pairs(g):
            slot = g % 2
            dst = pl.ds(base + g * G, G)
            return [(xo1.at[slot], x1_h.at[dst]), (xo2.at[slot], x2_h.at[dst])]

        def issue_s(g):
            for s, d in store_pairs(g):
                pltpu.async_copy(s, d, sem_s)

        def wait_s(g):
            for s, d in store_pairs(g):
                pltpu.make_async_copy(s, d, sem_s).wait()

        def compute(g):
            slot = g % 2

            def body_j(j, carry):
                for c in range(UD // L):
                    sl = pl.ds(c * L, L)
                    acc = rt[slot, j * MW, sl]
                    for r in range(1, MW):
                        acc = acc + rt[slot, j * MW + r, sl]
                    xo1[slot, j, pl.ds(c * L, L)] = acc * (1.0 / MW)
                for c in range(AD // L):
                    xo1[slot, j, pl.ds(UD + c * L, L)] = rg[slot, j, pl.ds(c * L, L)]
                for c in range(AD // L):
                    sl = pl.ds(c * L, L)
                    acc = ri[slot, j * MA, sl]
                    for r in range(1, MA):
                        acc = acc + ri[slot, j * MA + r, sl]
                    xo1[slot, j, pl.ds(UD + AD + c * L, L)] = acc * (1.0 / MA)
                for c in range(AD // L):
                    sl = pl.ds(c * L, L)
                    acc = rc[slot, j * MA, sl]
                    for r in range(1, MA):
                        acc = acc + rc[slot, j * MA + r, sl]
                    xo2[slot, j, pl.ds(c * L, L)] = acc * (1.0 / MA)
                for c in range(UD // L):
                    sl = pl.ds(c * L, L)
                    acc = ro[slot, j * MW, sl]
                    for r in range(1, MW):
                        acc = acc + ro[slot, j * MW + r, sl]
                    xo2[slot, j, pl.ds(AD + c * L, L)] = acc * (1.0 / MW)
                return carry

            lax.fori_loop(0, G, body_j, 0)

        issue_g(0)

        def loop_body(g, carry):
            issue_g(g)
            wait_g(g - 1)

            @pl.when(g >= 3)
            def _():
                wait_s(g - 3)

            compute(g - 1)
            issue_s(g - 1)
            return carry

        lax.fori_loop(1, NG, loop_body, 0)

        wait_g(NG - 1)
        wait_s(NG - 3)
        compute(NG - 1)
        issue_s(NG - 1)
        wait_s(NG - 2)
        wait_s(NG - 1)

    return k(tok, wt, tt, it, ct)


def _tc_head(x1, x2, coords, w1, w2, wc, b):
    BM = 2048

    def body(x1_ref, x2_ref, c_ref, w1_ref, w2_ref, wc_ref, b_ref, o_ref):
        acc = jnp.dot(x1_ref[...], w1_ref[...], preferred_element_type=jnp.float32)
        acc = acc + jnp.dot(x2_ref[...], w2_ref[...], preferred_element_type=jnp.float32)
        c = c_ref[...]
        acc = acc + c[:, 0:1] * wc_ref[0:1, :]
        acc = acc + c[:, 1:2] * wc_ref[1:2, :]
        acc = acc + c[:, 2:3] * wc_ref[2:3, :]
        o_ref[...] = acc + b_ref[...]

    return pl.pallas_call(
        body,
        grid=(B // BM,),
        in_specs=[
            pl.BlockSpec((BM, 128), lambda i: (i, 0)),
            pl.BlockSpec((BM, 128), lambda i: (i, 0)),
            pl.BlockSpec((BM, 3), lambda i: (i, 0)),
            pl.BlockSpec((128, DIM), lambda i: (0, 0)),
            pl.BlockSpec((128, DIM), lambda i: (0, 0)),
            pl.BlockSpec((3, DIM), lambda i: (0, 0)),
            pl.BlockSpec((1, DIM), lambda i: (0, 0)),
        ],
        out_specs=pl.BlockSpec((BM, DIM), lambda i: (i, 0)),
        out_shape=jax.ShapeDtypeStruct((B, DIM), jnp.float32),
    )(x1, x2, coords, w1, w2, wc, b)


def kernel(text_tokens, tags, id_tokens, class_tokens, other_tokens, coords,
           word_table, tag_table, id_table, class_table, fc_W, fc_b):
    tok = jnp.concatenate([
        text_tokens.astype(jnp.int32).reshape(-1),
        other_tokens.astype(jnp.int32).reshape(-1),
        id_tokens.astype(jnp.int32).reshape(-1),
        class_tokens.astype(jnp.int32).reshape(-1),
        tags.astype(jnp.int32),
    ])
    x1, x2 = _sc_embed(tok, word_table, tag_table, id_table, class_table)
    w1 = fc_W[:128]
    w2 = jnp.concatenate([fc_W[128:XCOLS], jnp.zeros((32, DIM), jnp.float32)])
    return _tc_head(x1, x2, coords, w1, w2, fc_W[XCOLS:],
                    fc_b.reshape(1, -1))


# speedup vs baseline: 12.7879x; 1.0107x over previous
import functools

import jax
import jax.numpy as jnp
from jax import lax
from jax.experimental import pallas as pl
from jax.experimental.pallas import tpu as pltpu
from jax.experimental.pallas import tpu_sc as plsc

B = 16384
MW = 20
MA = 10
UD = 64
AD = 32
DIM = 256
XCOLS = 2 * UD + 3 * AD

G = 8
L = 16
NS = 3


def _sc_embed(ti, tg, ii, ci, oi, wt, tt, it, ct):
    mesh = plsc.VectorSubcoreMesh(core_axis_name="c", subcore_axis_name="s")
    nw = mesh.num_cores * mesh.num_subcores
    R = B // nw
    NG = R // G

    @functools.partial(
        pl.kernel,
        out_type=(
            jax.ShapeDtypeStruct((B, 128), jnp.float32),
            jax.ShapeDtypeStruct((B, 128), jnp.float32),
        ),
        mesh=mesh,
        scratch_types=[
            pltpu.VMEM((R * MW,), jnp.int32),
            pltpu.VMEM((R,), jnp.int32),
            pltpu.VMEM((R * MA,), jnp.int32),
            pltpu.VMEM((R * MA,), jnp.int32),
            pltpu.VMEM((R * MW,), jnp.int32),
            pltpu.VMEM((NS, G * MW, UD), jnp.float32),
            pltpu.VMEM((NS, G, AD), jnp.float32),
            pltpu.VMEM((NS, G * MA, AD), jnp.float32),
            pltpu.VMEM((NS, G * MA, AD), jnp.float32),
            pltpu.VMEM((NS, G * MW, UD), jnp.float32),
            pltpu.VMEM((NS, G, 128), jnp.float32),
            pltpu.VMEM((NS, G, 128), jnp.float32),
            pltpu.SemaphoreType.DMA,
            pltpu.SemaphoreType.DMA,
        ],
        compiler_params=pltpu.CompilerParams(use_tc_tiling_on_sc=False),
    )
    def k(ti_h, tg_h, ii_h, ci_h, oi_h, wt_h, tt_h, it_h, ct_h, x1_h, x2_h,
          xt, xg, xi, xc, xo, rt, rg, ri, rc, ro, xo1, xo2, sem_g, sem_s):
        wid = lax.axis_index("s") * mesh.num_cores + lax.axis_index("c")
        base = wid * R

        pltpu.sync_copy(ti_h.at[pl.ds(base * MW, R * MW)], xt)
        pltpu.sync_copy(tg_h.at[pl.ds(base, R)], xg)
        pltpu.sync_copy(ii_h.at[pl.ds(base * MA, R * MA)], xi)
        pltpu.sync_copy(ci_h.at[pl.ds(base * MA, R * MA)], xc)
        pltpu.sync_copy(oi_h.at[pl.ds(base * MW, R * MW)], xo)

        zero = jnp.zeros((L,), jnp.float32)
        for s in range(NS):
            for j in range(G):
                xo2[s, j, pl.ds(96, L)] = zero
                xo2[s, j, pl.ds(96 + L, L)] = zero

        H = G * MW // 2

        def gather_pairs(g):
            slot = g % NS
            tw = g * G * MW
            ta = g * G * MA
            return [
                (wt_h.at[xt.at[pl.ds(tw, H)]], rt.at[slot, pl.ds(0, H)]),
                (wt_h.at[xt.at[pl.ds(tw + H, H)]], rt.at[slot, pl.ds(H, H)]),
                (wt_h.at[xo.at[pl.ds(tw, H)]], ro.at[slot, pl.ds(0, H)]),
                (wt_h.at[xo.at[pl.ds(tw + H, H)]], ro.at[slot, pl.ds(H, H)]),
                (it_h.at[xi.at[pl.ds(ta, G * MA)]], ri.at[slot]),
                (ct_h.at[xc.at[pl.ds(ta, G * MA)]], rc.at[slot]),
                (tt_h.at[xg.at[pl.ds(g * G, G)]], rg.at[slot]),
            ]

        def issue_g(g):
            for s, d in gather_pairs(g):
                pltpu.async_copy(s, d, sem_g)

        def wait_g(g):
            for s, d in gather_pairs(g):
                pltpu.make_async_copy(s, d, sem_g).wait()

        def store_pairs(g):
            slot = g % NS
            dst = pl.ds(base + g * G, G)
            return [(xo1.at[slot], x1_h.at[dst]), (xo2.at[slot], x2_h.at[dst])]

        def issue_s(g):
            for s, d in store_pairs(g):
                pltpu.async_copy(s, d, sem_s)

        def wait_s(g):
            for s, d in store_pairs(g):
                pltpu.make_async_copy(s, d, sem_s).wait()

        def compute(g):
            slot = g % NS

            def body_j(j, carry):
                for c in range(UD // L):
                    sl = pl.ds(c * L, L)
                    acc = rt[slot, j * MW, sl]
                    for r in range(1, MW):
                        acc = acc + rt[slot, j * MW + r, sl]
                    xo1[slot, j, pl.ds(c * L, L)] = acc * (1.0 / MW)
                for c in range(AD // L):
                    xo1[slot, j, pl.ds(UD + c * L, L)] = rg[slot, j, pl.ds(c * L, L)]
                for c in range(AD // L):
                    sl = pl.ds(c * L, L)
                    acc = ri[slot, j * MA, sl]
                    for r in range(1, MA):
                        acc = acc + ri[slot, j * MA + r, sl]
                    xo1[slot, j, pl.ds(UD + AD + c * L, L)] = acc * (1.0 / MA)
                for c in range(AD // L):
                    sl = pl.ds(c * L, L)
                    acc = rc[slot, j * MA, sl]
                    for r in range(1, MA):
                        acc = acc + rc[slot, j * MA + r, sl]
                    xo2[slot, j, pl.ds(c * L, L)] = acc * (1.0 / MA)
                for c in range(UD // L):
                    sl = pl.ds(c * L, L)
                    acc = ro[slot, j * MW, sl]
                    for r in range(1, MW):
                        acc = acc + ro[slot, j * MW + r, sl]
                    xo2[slot, j, pl.ds(AD + c * L, L)] = acc * (1.0 / MW)
                return carry

            lax.fori_loop(0, G, body_j, 0)

        issue_g(0)
        issue_g(1)

        def loop_body(g, carry):
            issue_g(g)
            wait_g(g - 2)

            @pl.when(g >= 5)
            def _():
                wait_s(g - 5)

            compute(g - 2)
            issue_s(g - 2)
            return carry

        lax.fori_loop(2, NG, loop_body, 0)

        wait_g(NG - 2)
        wait_s(NG - 5)
        compute(NG - 2)
        issue_s(NG - 2)
        wait_g(NG - 1)
        wait_s(NG - 4)
        compute(NG - 1)
        issue_s(NG - 1)
        wait_s(NG - 3)
        wait_s(NG - 2)
        wait_s(NG - 1)

    return k(ti, tg, ii, ci, oi, wt, tt, it, ct)


def _tc_head(x1, x2, coords, w1, w2, wc, b):
    BM = 2048

    def body(x1_ref, x2_ref, c_ref, w1_ref, w2_ref, wc_ref, b_ref, o_ref):
        acc = jnp.dot(x1_ref[...], w1_ref[...], preferred_element_type=jnp.float32)
        acc = acc + jnp.dot(x2_ref[...], w2_ref[...], preferred_element_type=jnp.float32)
        c = c_ref[...]
        acc = acc + c[:, 0:1] * wc_ref[0:1, :]
        acc = acc + c[:, 1:2] * wc_ref[1:2, :]
        acc = acc + c[:, 2:3] * wc_ref[2:3, :]
        o_ref[...] = acc + b_ref[...]

    return pl.pallas_call(
        body,
        grid=(B // BM,),
        in_specs=[
            pl.BlockSpec((BM, 128), lambda i: (i, 0)),
            pl.BlockSpec((BM, 128), lambda i: (i, 0)),
            pl.BlockSpec((BM, 3), lambda i: (i, 0)),
            pl.BlockSpec((128, DIM), lambda i: (0, 0)),
            pl.BlockSpec((128, DIM), lambda i: (0, 0)),
            pl.BlockSpec((3, DIM), lambda i: (0, 0)),
            pl.BlockSpec((1, DIM), lambda i: (0, 0)),
        ],
        out_specs=pl.BlockSpec((BM, DIM), lambda i: (i, 0)),
        out_shape=jax.ShapeDtypeStruct((B, DIM), jnp.float32),
    )(x1, x2, coords, w1, w2, wc, b)


def kernel(text_tokens, tags, id_tokens, class_tokens, other_tokens, coords,
           word_table, tag_table, id_table, class_table, fc_W, fc_b):
    ti = text_tokens.astype(jnp.int32).reshape(-1)
    tg = tags.astype(jnp.int32)
    ii = id_tokens.astype(jnp.int32).reshape(-1)
    ci = class_tokens.astype(jnp.int32).reshape(-1)
    oi = other_tokens.astype(jnp.int32).reshape(-1)
    x1, x2 = _sc_embed(ti, tg, ii, ci, oi, word_table, tag_table, id_table,
                       class_table)
    w1 = fc_W[:128]
    w2 = jnp.concatenate([fc_W[128:XCOLS], jnp.zeros((32, DIM), jnp.float32)])
    return _tc_head(x1, x2, coords, w1, w2, fc_W[XCOLS:],
                    fc_b.reshape(1, -1))


# speedup vs baseline: 12.8254x; 1.0029x over previous
import functools

import jax
import jax.numpy as jnp
from jax import lax
from jax.experimental import pallas as pl
from jax.experimental.pallas import tpu as pltpu
from jax.experimental.pallas import tpu_sc as plsc

B = 16384
MW = 20
MA = 10
UD = 64
AD = 32
DIM = 256
XCOLS = 2 * UD + 3 * AD

G = 8
L = 16
NS = 3


def _sc_embed(ti, tg, ii, ci, oi, wt, tt, it, ct):
    mesh = plsc.VectorSubcoreMesh(core_axis_name="c", subcore_axis_name="s")
    nw = mesh.num_cores * mesh.num_subcores
    R = B // nw
    NG = R // G

    @functools.partial(
        pl.kernel,
        out_type=(
            jax.ShapeDtypeStruct((B, 128), jnp.float32),
            jax.ShapeDtypeStruct((B, 128), jnp.float32),
        ),
        mesh=mesh,
        scratch_types=[
            pltpu.VMEM((R * MW,), jnp.int32),
            pltpu.VMEM((R,), jnp.int32),
            pltpu.VMEM((R * MA,), jnp.int32),
            pltpu.VMEM((R * MA,), jnp.int32),
            pltpu.VMEM((R * MW,), jnp.int32),
            pltpu.VMEM((NS, G * MW, UD), jnp.float32),
            pltpu.VMEM((NS, G, AD), jnp.float32),
            pltpu.VMEM((NS, G * MA, AD), jnp.float32),
            pltpu.VMEM((NS, G * MA, AD), jnp.float32),
            pltpu.VMEM((NS, G * MW, UD), jnp.float32),
            pltpu.VMEM((NS, G, 128), jnp.float32),
            pltpu.VMEM((NS, G, 128), jnp.float32),
            pltpu.SemaphoreType.DMA,
            pltpu.SemaphoreType.DMA,
        ],
        compiler_params=pltpu.CompilerParams(use_tc_tiling_on_sc=False),
    )
    def k(ti_h, tg_h, ii_h, ci_h, oi_h, wt_h, tt_h, it_h, ct_h, x1_h, x2_h,
          xt, xg, xi, xc, xo, rt, rg, ri, rc, ro, xo1, xo2, sem_g, sem_s):
        wid = lax.axis_index("s") * mesh.num_cores + lax.axis_index("c")
        base = wid * R

        pltpu.sync_copy(ti_h.at[pl.ds(base * MW, R * MW)], xt)
        pltpu.sync_copy(tg_h.at[pl.ds(base, R)], xg)
        pltpu.sync_copy(ii_h.at[pl.ds(base * MA, R * MA)], xi)
        pltpu.sync_copy(ci_h.at[pl.ds(base * MA, R * MA)], xc)
        pltpu.sync_copy(oi_h.at[pl.ds(base * MW, R * MW)], xo)

        zero = jnp.zeros((L,), jnp.float32)
        for s in range(NS):
            for j in range(G):
                xo2[s, j, pl.ds(96, L)] = zero
                xo2[s, j, pl.ds(96 + L, L)] = zero

        H = G * MW // 2

        def gather_pairs(g):
            slot = g % NS
            tw = g * G * MW
            ta = g * G * MA
            return [
                (wt_h.at[xt.at[pl.ds(tw, G * MW)]], rt.at[slot]),
                (wt_h.at[xo.at[pl.ds(tw, G * MW)]], ro.at[slot]),
                (it_h.at[xi.at[pl.ds(ta, G * MA)]], ri.at[slot]),
                (ct_h.at[xc.at[pl.ds(ta, G * MA)]], rc.at[slot]),
                (tt_h.at[xg.at[pl.ds(g * G, G)]], rg.at[slot]),
            ]

        def issue_g(g):
            for s, d in gather_pairs(g):
                pltpu.async_copy(s, d, sem_g)

        def wait_g(g):
            for s, d in gather_pairs(g):
                pltpu.make_async_copy(s, d, sem_g).wait()

        def store_pairs(g):
            slot = g % NS
            dst = pl.ds(base + g * G, G)
            return [(xo1.at[slot], x1_h.at[dst]), (xo2.at[slot], x2_h.at[dst])]

        def issue_s(g):
            for s, d in store_pairs(g):
                pltpu.async_copy(s, d, sem_s)

        def wait_s(g):
            for s, d in store_pairs(g):
                pltpu.make_async_copy(s, d, sem_s).wait()

        def compute(g):
            slot = g % NS

            def body_j(j, carry):
                for c in range(UD // L):
                    sl = pl.ds(c * L, L)
                    acc = rt[slot, j * MW, sl]
                    for r in range(1, MW):
                        acc = acc + rt[slot, j * MW + r, sl]
                    xo1[slot, j, pl.ds(c * L, L)] = acc * (1.0 / MW)
                for c in range(AD // L):
                    xo1[slot, j, pl.ds(UD + c * L, L)] = rg[slot, j, pl.ds(c * L, L)]
                for c in range(AD // L):
                    sl = pl.ds(c * L, L)
                    acc = ri[slot, j * MA, sl]
                    for r in range(1, MA):
                        acc = acc + ri[slot, j * MA + r, sl]
                    xo1[slot, j, pl.ds(UD + AD + c * L, L)] = acc * (1.0 / MA)
                for c in range(AD // L):
                    sl = pl.ds(c * L, L)
                    acc = rc[slot, j * MA, sl]
                    for r in range(1, MA):
                        acc = acc + rc[slot, j * MA + r, sl]
                    xo2[slot, j, pl.ds(c * L, L)] = acc * (1.0 / MA)
                for c in range(UD // L):
                    sl = pl.ds(c * L, L)
                    acc = ro[slot, j * MW, sl]
                    for r in range(1, MW):
                        acc = acc + ro[slot, j * MW + r, sl]
                    xo2[slot, j, pl.ds(AD + c * L, L)] = acc * (1.0 / MW)
                return carry

            lax.fori_loop(0, G, body_j, 0)

        issue_g(0)
        issue_g(1)

        def loop_body(g, carry):
            issue_g(g)
            wait_g(g - 2)

            @pl.when(g >= 5)
            def _():
                wait_s(g - 5)

            compute(g - 2)
            issue_s(g - 2)
            return carry

        lax.fori_loop(2, NG, loop_body, 0)

        wait_g(NG - 2)
        wait_s(NG - 5)
        compute(NG - 2)
        issue_s(NG - 2)
        wait_g(NG - 1)
        wait_s(NG - 4)
        compute(NG - 1)
        issue_s(NG - 1)
        wait_s(NG - 3)
        wait_s(NG - 2)
        wait_s(NG - 1)

    return k(ti, tg, ii, ci, oi, wt, tt, it, ct)


def _tc_head(x1, x2, coords, w1, w2, wc, b):
    BM = 2048

    def body(x1_ref, x2_ref, c_ref, w1_ref, w2_ref, wc_ref, b_ref, o_ref):
        acc = jnp.dot(x1_ref[...], w1_ref[...], preferred_element_type=jnp.float32)
        acc = acc + jnp.dot(x2_ref[...], w2_ref[...], preferred_element_type=jnp.float32)
        c = c_ref[...]
        acc = acc + c[:, 0:1] * wc_ref[0:1, :]
        acc = acc + c[:, 1:2] * wc_ref[1:2, :]
        acc = acc + c[:, 2:3] * wc_ref[2:3, :]
        o_ref[...] = acc + b_ref[...]

    return pl.pallas_call(
        body,
        grid=(B // BM,),
        in_specs=[
            pl.BlockSpec((BM, 128), lambda i: (i, 0)),
            pl.BlockSpec((BM, 128), lambda i: (i, 0)),
            pl.BlockSpec((BM, 3), lambda i: (i, 0)),
            pl.BlockSpec((128, DIM), lambda i: (0, 0)),
            pl.BlockSpec((128, DIM), lambda i: (0, 0)),
            pl.BlockSpec((3, DIM), lambda i: (0, 0)),
            pl.BlockSpec((1, DIM), lambda i: (0, 0)),
        ],
        out_specs=pl.BlockSpec((BM, DIM), lambda i: (i, 0)),
        out_shape=jax.ShapeDtypeStruct((B, DIM), jnp.float32),
    )(x1, x2, coords, w1, w2, wc, b)


def kernel(text_tokens, tags, id_tokens, class_tokens, other_tokens, coords,
           word_table, tag_table, id_table, class_table, fc_W, fc_b):
    ti = text_tokens.astype(jnp.int32).reshape(-1)
    tg = tags.astype(jnp.int32)
    ii = id_tokens.astype(jnp.int32).reshape(-1)
    ci = class_tokens.astype(jnp.int32).reshape(-1)
    oi = other_tokens.astype(jnp.int32).reshape(-1)
    x1, x2 = _sc_embed(ti, tg, ii, ci, oi, word_table, tag_table, id_table,
                       class_table)
    w1 = fc_W[:128]
    w2 = jnp.concatenate([fc_W[128:XCOLS], jnp.zeros((32, DIM), jnp.float32)])
    return _tc_head(x1, x2, coords, w1, w2, fc_W[XCOLS:],
                    fc_b.reshape(1, -1))
